# Initial kernel scaffold; baseline (speedup 1.0000x reference)
import jax
import jax.numpy as jnp
from jax.experimental import pallas as pl


def kernel(x, edge_index, W_in, b_in, ln_in_s, ln_in_b, W_tm, b_tm, ln1_s, ln1_b, ln2_s, ln2_b, W_out, b_out):
    raise NotImplementedError("write your pallas kernel here")


# speedup vs baseline: 7.3911x; 7.3911x over previous
import functools

import jax
import jax.numpy as jnp
from jax import lax
from jax.experimental import pallas as pl
from jax.experimental.pallas import tpu as pltpu
from jax.experimental.pallas import tpu_sc as plsc

N = 10000
NP = 10240
E = 320000
D = 128
EPS = 1e-5

NC = 2
NS = 16
NW = NC * NS
EPW = E // NW
K = 80
NCH = EPW // K
RPT = NP // NS
ZR = 128
CC = 2000
BN = 1024

_MESH = plsc.VectorSubcoreMesh(
    core_axis_name="c", subcore_axis_name="s", num_cores=NC, num_subcores=NS)


def _counts_body(src_hbm, dst_hbm, out_hbm, sidx, didx, deg, slf):
    c = lax.axis_index("c")
    s = lax.axis_index("s")
    w = s * NC + c
    base = w * EPW
    zeros = jnp.zeros((16,), jnp.float32)
    ones = jnp.ones((16,), jnp.float32)

    def zloop(i, _):
        deg[pl.ds(i * 16, 16)] = zeros
        slf[pl.ds(i * 16, 16)] = zeros
        return ()
    lax.fori_loop(0, NP // 16, zloop, ())

    def chunk(i, _):
        pltpu.sync_copy(src_hbm.at[pl.ds(base + i * CC, CC)], sidx)
        pltpu.sync_copy(dst_hbm.at[pl.ds(base + i * CC, CC)], didx)

        def inner(j, _):
            sv = sidx[pl.ds(j * 16, 16)]
            dv = didx[pl.ds(j * 16, 16)]
            m = sv != dv
            plsc.addupdate_scatter(deg, [dv], ones, mask=m)
            plsc.addupdate_scatter(slf, [dv], ones, mask=jnp.logical_not(m))
            return ()
        lax.fori_loop(0, CC // 16, inner, ())
        return ()
    lax.fori_loop(0, EPW // CC, chunk, ())

    pltpu.sync_copy(deg, out_hbm.at[w, 0])
    pltpu.sync_copy(slf, out_hbm.at[w, 1])


_sc_counts = pl.kernel(
    _counts_body,
    out_type=jax.ShapeDtypeStruct((NW, 2, NP), jnp.float32),
    mesh=_MESH,
    scratch_types=[
        pltpu.VMEM((CC,), jnp.int32),
        pltpu.VMEM((CC,), jnp.int32),
        pltpu.VMEM((NP,), jnp.float32),
        pltpu.VMEM((NP,), jnp.float32),
    ],
    compiler_params=pltpu.CompilerParams(needs_layout_passes=False),
    name="sc_edge_counts",
)


def _agg_body(h_hbm, src_hbm, dst_hbm, out_hbm, sidx, didx, rows, zbuf, acc, sem):
    c = lax.axis_index("c")
    s = lax.axis_index("s")
    w = s * NC + c
    base = w * EPW
    zeros = jnp.zeros((16,), jnp.float32)

    def z1(i, _):
        zbuf[i // 8, pl.ds((i % 8) * 16, 16)] = zeros
        return ()
    lax.fori_loop(0, ZR * 8, z1, ())

    r0 = s * RPT

    def zc(i, _):
        pltpu.sync_copy(zbuf, acc.at[pl.ds(r0 + i * ZR, ZR)])
        return ()
    lax.fori_loop(0, RPT // ZR, zc, ())
    plsc.subcore_barrier()

    def chunk(i, _):
        off = base + i * K
        pltpu.sync_copy(src_hbm.at[pl.ds(off, K)], sidx.at[0])
        pltpu.sync_copy(dst_hbm.at[pl.ds(off, K)], didx.at[0])
        pltpu.async_copy(h_hbm.at[sidx.at[0]], rows, sem).wait()
        pltpu.sync_copy(rows, acc.at[didx.at[0]], add=True)
        return ()
    lax.fori_loop(0, NCH, chunk, ())
    plsc.subcore_barrier()

    pltpu.sync_copy(acc.at[pl.ds(r0, RPT)], out_hbm.at[c, pl.ds(r0, RPT)])


_sc_agg = pl.kernel(
    _agg_body,
    out_type=jax.ShapeDtypeStruct((NC, NP, D), jnp.float32),
    mesh=_MESH,
    scratch_types=[
        pltpu.VMEM((1, K), jnp.int32),
        pltpu.VMEM((1, K), jnp.int32),
        pltpu.VMEM((K, D), jnp.float32),
        pltpu.VMEM((ZR, D), jnp.float32),
        pltpu.VMEM_SHARED((NP, D), jnp.float32),
        pltpu.SemaphoreType.DMA,
    ],
    name="sc_segment_sum",
)


def _ln_rows(h, s_row, b_row):
    mu = jnp.mean(h, axis=-1, keepdims=True)
    var = jnp.mean((h - mu) ** 2, axis=-1, keepdims=True)
    return (h - mu) / jnp.sqrt(var + EPS) * s_row + b_row


def _in_body(x_ref, w_ref, b_ref, ls_ref, lb_ref, o_ref):
    h = jnp.dot(x_ref[...], w_ref[...], preferred_element_type=jnp.float32)
    h = jnp.maximum(h + b_ref[...], 0.0)
    o_ref[...] = _ln_rows(h, ls_ref[...], lb_ref[...])


def _tc_in(x, W_in, b_in, ln_s, ln_b):
    return pl.pallas_call(
        _in_body,
        grid=(NP // BN,),
        in_specs=[
            pl.BlockSpec((BN, D), lambda i: (i, 0)),
            pl.BlockSpec((D, D), lambda i: (0, 0)),
            pl.BlockSpec((1, D), lambda i: (0, 0)),
            pl.BlockSpec((1, D), lambda i: (0, 0)),
            pl.BlockSpec((1, D), lambda i: (0, 0)),
        ],
        out_specs=pl.BlockSpec((BN, D), lambda i: (i, 0)),
        out_shape=jax.ShapeDtypeStruct((NP, D), jnp.float32),
    )(x, W_in, b_in.reshape(1, D), ln_s.reshape(1, D), ln_b.reshape(1, D))


def _gate_body(h_ref, p0_ref, p1_ref, cnt_ref, w1_ref, w2_ref, bt_ref,
               ls_ref, lb_ref, wo_ref, bo_ref, o_ref, *, fuse_out):
    h = h_ref[...]
    colsum = jnp.sum(cnt_ref[...], axis=0)
    cnt = colsum[0][:, None] + 1.0
    corr = 1.0 - colsum[1][:, None]
    ssum = p0_ref[...] + p1_ref[...] + corr * h
    m = ssum / cnt
    z = (jnp.dot(h, w1_ref[...], preferred_element_type=jnp.float32)
         + jnp.dot(m, w2_ref[...], preferred_element_type=jnp.float32)
         + bt_ref[...])
    tm = jax.nn.sigmoid(z)
    hn = _ln_rows(h * tm + m * (1.0 - tm), ls_ref[...], lb_ref[...])
    if fuse_out:
        o_ref[...] = (jnp.dot(hn, wo_ref[...], preferred_element_type=jnp.float32)
                      + bo_ref[...])
    else:
        o_ref[...] = hn


def _tc_gate(h, p0, p1, cntp, W1, W2, bt, ln_s, ln_b, W_out, b_out, fuse_out):
    return pl.pallas_call(
        functools.partial(_gate_body, fuse_out=fuse_out),
        grid=(NP // BN,),
        in_specs=[
            pl.BlockSpec((BN, D), lambda i: (i, 0)),
            pl.BlockSpec((BN, D), lambda i: (i, 0)),
            pl.BlockSpec((BN, D), lambda i: (i, 0)),
            pl.BlockSpec((NW, 2, BN), lambda i: (0, 0, i)),
            pl.BlockSpec((D, D), lambda i: (0, 0)),
            pl.BlockSpec((D, D), lambda i: (0, 0)),
            pl.BlockSpec((1, D), lambda i: (0, 0)),
            pl.BlockSpec((1, D), lambda i: (0, 0)),
            pl.BlockSpec((1, D), lambda i: (0, 0)),
            pl.BlockSpec((D, D), lambda i: (0, 0)),
            pl.BlockSpec((1, D), lambda i: (0, 0)),
        ],
        out_specs=pl.BlockSpec((BN, D), lambda i: (i, 0)),
        out_shape=jax.ShapeDtypeStruct((NP, D), jnp.float32),
    )(h, p0, p1, cntp, W1, W2, bt.reshape(1, D), ln_s.reshape(1, D),
      ln_b.reshape(1, D), W_out, b_out.reshape(1, D))


def kernel(x, edge_index, W_in, b_in, ln_in_s, ln_in_b, W_tm, b_tm,
           ln1_s, ln1_b, ln2_s, ln2_b, W_out, b_out):
    src = edge_index[0]
    dst = edge_index[1]
    x = jnp.pad(x, ((0, NP - N), (0, 0)))
    rep = D // W_tm.shape[1]
    W_exp = jnp.repeat(W_tm, rep, axis=1)
    W1 = W_exp[:D]
    W2 = W_exp[D:]
    bt = jnp.repeat(b_tm, rep)

    h0 = _tc_in(x, W_in, b_in, ln_in_s, ln_in_b)
    cntp = _sc_counts(src, dst)

    p = _sc_agg(h0, src, dst)
    h1 = _tc_gate(h0, p[0], p[1], cntp, W1, W2, bt, ln1_s, ln1_b,
                  W_out, b_out, fuse_out=False)
    p2 = _sc_agg(h1, src, dst)
    out = _tc_gate(h1, p2[0], p2[1], cntp, W1, W2, bt, ln2_s, ln2_b,
                   W_out, b_out, fuse_out=True)
    return out[:N]


# speedup vs baseline: 14.5600x; 1.9699x over previous
import functools

import jax
import jax.numpy as jnp
from jax import lax
from jax.experimental import pallas as pl
from jax.experimental.pallas import tpu as pltpu
from jax.experimental.pallas import tpu_sc as plsc

N = 10000
NP = 10240
E = 320000
D = 128
EPS = 1e-5

NC = 2
NS = 16
NW = NC * NS
EPW = E // NW
KP = 128
NCHP = 79
EPWP = NCHP * KP
EPAD = NW * EPWP
RPT = NP // NS
ZR = 128
CC = 2000
BN = 1024

_MESH = plsc.VectorSubcoreMesh(
    core_axis_name="c", subcore_axis_name="s", num_cores=NC, num_subcores=NS)


def _counts_body(src_hbm, dst_hbm, out_hbm, sidx, didx, deg, slf):
    c = lax.axis_index("c")
    s = lax.axis_index("s")
    w = s * NC + c
    base = w * EPW
    zeros = jnp.zeros((16,), jnp.float32)
    ones = jnp.ones((16,), jnp.float32)

    def zloop(i, _):
        deg[pl.ds(i * 16, 16)] = zeros
        slf[pl.ds(i * 16, 16)] = zeros
        return ()
    lax.fori_loop(0, NP // 16, zloop, ())

    def chunk(i, _):
        pltpu.sync_copy(src_hbm.at[pl.ds(base + i * CC, CC)], sidx)
        pltpu.sync_copy(dst_hbm.at[pl.ds(base + i * CC, CC)], didx)

        def inner(j, _):
            sv = sidx[pl.ds(j * 16, 16)]
            dv = didx[pl.ds(j * 16, 16)]
            m = sv != dv
            plsc.addupdate_scatter(deg, [dv], ones, mask=m)
            plsc.addupdate_scatter(slf, [dv], ones, mask=jnp.logical_not(m))
            return ()
        lax.fori_loop(0, CC // 16, inner, ())
        return ()
    lax.fori_loop(0, EPW // CC, chunk, ())

    pltpu.sync_copy(deg, out_hbm.at[w, 0])
    pltpu.sync_copy(slf, out_hbm.at[w, 1])


_sc_counts = pl.kernel(
    _counts_body,
    out_type=jax.ShapeDtypeStruct((NW, 2, NP), jnp.float32),
    mesh=_MESH,
    scratch_types=[
        pltpu.VMEM((CC,), jnp.int32),
        pltpu.VMEM((CC,), jnp.int32),
        pltpu.VMEM((NP,), jnp.float32),
        pltpu.VMEM((NP,), jnp.float32),
    ],
    compiler_params=pltpu.CompilerParams(needs_layout_passes=False),
    name="sc_edge_counts",
)


def _agg_body(h_hbm, src_hbm, dst_hbm, out_hbm, sidx, didx, rows0, rows1,
              acc, isem, gsem0, gsem1):
    c = lax.axis_index("c")
    s = lax.axis_index("s")
    w = s * NC + c
    zeros = jnp.zeros((16,), jnp.float32)

    def z1(i, _):
        rows0[i // 8, pl.ds((i % 8) * 16, 16)] = zeros
        return ()
    lax.fori_loop(0, KP * 8, z1, ())

    r0 = s * RPT

    def zc(i, _):
        pltpu.sync_copy(rows0, acc.at[pl.ds(r0 + i * KP, KP)])
        return ()
    lax.fori_loop(0, RPT // KP, zc, ())
    plsc.subcore_barrier()

    rows = (rows0, rows1)
    gsems = (gsem0, gsem1)

    def idx_copy(i, b):
        c0 = pltpu.async_copy(src_hbm.at[w, i], sidx.at[b], isem)
        c1 = pltpu.async_copy(dst_hbm.at[w, i], didx.at[b], isem)
        return c0, c1

    i0, i1 = idx_copy(0, 0)
    i0.wait()
    i1.wait()
    pltpu.async_copy(h_hbm.at[sidx.at[0]], rows0, gsem0)
    idx_copy(1, 1)

    def step(ii, _):
        for b in range(2):
            i = 2 * ii + b
            o = 1 - b

            @pl.when(i < NCHP)
            def _(i=i, b=b, o=o):
                @pl.when(i + 1 < NCHP)
                def _():
                    pltpu.make_async_copy(src_hbm.at[w, i + 1], sidx.at[o],
                                          isem).wait()
                    pltpu.make_async_copy(dst_hbm.at[w, i + 1], didx.at[o],
                                          isem).wait()
                pltpu.make_async_copy(h_hbm.at[sidx.at[b]], rows[b],
                                      gsems[b]).wait()

                @pl.when(i + 1 < NCHP)
                def _():
                    pltpu.async_copy(h_hbm.at[sidx.at[o]], rows[o], gsems[o])
                pltpu.sync_copy(rows[b], acc.at[didx.at[b]], add=True)

                @pl.when(i + 2 < NCHP)
                def _():
                    idx_copy(i + 2, b)
        return ()
    lax.fori_loop(0, (NCHP + 1) // 2, step, ())
    plsc.subcore_barrier()

    pltpu.sync_copy(acc.at[pl.ds(r0, RPT)], out_hbm.at[c, pl.ds(r0, RPT)])


_sc_agg = pl.kernel(
    _agg_body,
    out_type=jax.ShapeDtypeStruct((NC, NP, D), jnp.float32),
    mesh=_MESH,
    scratch_types=[
        pltpu.VMEM((2, KP), jnp.int32),
        pltpu.VMEM((2, KP), jnp.int32),
        pltpu.VMEM((KP, D), jnp.float32),
        pltpu.VMEM((KP, D), jnp.float32),
        pltpu.VMEM_SHARED((NP, D), jnp.float32),
        pltpu.SemaphoreType.DMA,
        pltpu.SemaphoreType.DMA,
        pltpu.SemaphoreType.DMA,
    ],
    name="sc_segment_sum",
)


def _ln_rows(h, s_row, b_row):
    mu = jnp.mean(h, axis=-1, keepdims=True)
    var = jnp.mean((h - mu) ** 2, axis=-1, keepdims=True)
    return (h - mu) / jnp.sqrt(var + EPS) * s_row + b_row


def _in_body(x_ref, w_ref, b_ref, ls_ref, lb_ref, o_ref):
    h = jnp.dot(x_ref[...], w_ref[...], preferred_element_type=jnp.float32)
    h = jnp.maximum(h + b_ref[...], 0.0)
    o_ref[...] = _ln_rows(h, ls_ref[...], lb_ref[...])


def _tc_in(x, W_in, b_in, ln_s, ln_b):
    return pl.pallas_call(
        _in_body,
        grid=(NP // BN,),
        in_specs=[
            pl.BlockSpec((BN, D), lambda i: (i, 0)),
            pl.BlockSpec((D, D), lambda i: (0, 0)),
            pl.BlockSpec((1, D), lambda i: (0, 0)),
            pl.BlockSpec((1, D), lambda i: (0, 0)),
            pl.BlockSpec((1, D), lambda i: (0, 0)),
        ],
        out_specs=pl.BlockSpec((BN, D), lambda i: (i, 0)),
        out_shape=jax.ShapeDtypeStruct((NP, D), jnp.float32),
    )(x, W_in, b_in.reshape(1, D), ln_s.reshape(1, D), ln_b.reshape(1, D))


def _gate_body(h_ref, p0_ref, p1_ref, cnt_ref, w1_ref, w2_ref, bt_ref,
               ls_ref, lb_ref, wo_ref, bo_ref, o_ref, *, fuse_out):
    h = h_ref[...]
    colsum = jnp.sum(cnt_ref[...], axis=0)
    cnt = colsum[0][:, None] + 1.0
    corr = 1.0 - colsum[1][:, None]
    ssum = p0_ref[...] + p1_ref[...] + corr * h
    m = ssum / cnt
    z = (jnp.dot(h, w1_ref[...], preferred_element_type=jnp.float32)
         + jnp.dot(m, w2_ref[...], preferred_element_type=jnp.float32)
         + bt_ref[...])
    tm = jax.nn.sigmoid(z)
    hn = _ln_rows(h * tm + m * (1.0 - tm), ls_ref[...], lb_ref[...])
    if fuse_out:
        o_ref[...] = (jnp.dot(hn, wo_ref[...], preferred_element_type=jnp.float32)
                      + bo_ref[...])
    else:
        o_ref[...] = hn


def _tc_gate(h, p0, p1, cntp, W1, W2, bt, ln_s, ln_b, W_out, b_out, fuse_out):
    return pl.pallas_call(
        functools.partial(_gate_body, fuse_out=fuse_out),
        grid=(NP // BN,),
        in_specs=[
            pl.BlockSpec((BN, D), lambda i: (i, 0)),
            pl.BlockSpec((BN, D), lambda i: (i, 0)),
            pl.BlockSpec((BN, D), lambda i: (i, 0)),
            pl.BlockSpec((NW, 2, BN), lambda i: (0, 0, i)),
            pl.BlockSpec((D, D), lambda i: (0, 0)),
            pl.BlockSpec((D, D), lambda i: (0, 0)),
            pl.BlockSpec((1, D), lambda i: (0, 0)),
            pl.BlockSpec((1, D), lambda i: (0, 0)),
            pl.BlockSpec((1, D), lambda i: (0, 0)),
            pl.BlockSpec((D, D), lambda i: (0, 0)),
            pl.BlockSpec((1, D), lambda i: (0, 0)),
        ],
        out_specs=pl.BlockSpec((BN, D), lambda i: (i, 0)),
        out_shape=jax.ShapeDtypeStruct((NP, D), jnp.float32),
    )(h, p0, p1, cntp, W1, W2, bt.reshape(1, D), ln_s.reshape(1, D),
      ln_b.reshape(1, D), W_out, b_out.reshape(1, D))


def kernel(x, edge_index, W_in, b_in, ln_in_s, ln_in_b, W_tm, b_tm,
           ln1_s, ln1_b, ln2_s, ln2_b, W_out, b_out):
    src = edge_index[0]
    dst = edge_index[1]
    x = jnp.pad(x, ((0, NP - N), (0, 0)))
    padidx = N + (jnp.arange(EPAD - E, dtype=jnp.int32) % (NP - N))
    src_r = jnp.concatenate([src, padidx]).reshape(NW, NCHP, KP)
    dst_r = jnp.concatenate([dst, padidx]).reshape(NW, NCHP, KP)
    rep = D // W_tm.shape[1]
    W_exp = jnp.repeat(W_tm, rep, axis=1)
    W1 = W_exp[:D]
    W2 = W_exp[D:]
    bt = jnp.repeat(b_tm, rep)

    h0 = _tc_in(x, W_in, b_in, ln_in_s, ln_in_b)
    cntp = _sc_counts(src, dst)

    p = _sc_agg(h0, src_r, dst_r)
    h1 = _tc_gate(h0, p[0], p[1], cntp, W1, W2, bt, ln1_s, ln1_b,
                  W_out, b_out, fuse_out=False)
    p2 = _sc_agg(h1, src_r, dst_r)
    out = _tc_gate(h1, p2[0], p2[1], cntp, W1, W2, bt, ln2_s, ln2_b,
                   W_out, b_out, fuse_out=True)
    return out[:N]


# speedup vs baseline: 17.8874x; 1.2285x over previous
import functools

import jax
import jax.numpy as jnp
from jax import lax
from jax.experimental import pallas as pl
from jax.experimental.pallas import tpu as pltpu
from jax.experimental.pallas import tpu_sc as plsc

N = 10000
NP = 10240
E = 320000
D = 128
EPS = 1e-5

NC = 2
NS = 16
NW = NC * NS
EPW = E // NW
KP = 120
NCHP = 85
EPWP = NCHP * KP
EPAD = NW * EPWP
GRP = 6
RPT = NP // NS
ZR = 128
CC = 2000
BN = 1024

_MESH = plsc.VectorSubcoreMesh(
    core_axis_name="c", subcore_axis_name="s", num_cores=NC, num_subcores=NS)


def _counts_body(src_hbm, dst_hbm, out_hbm, sidx, didx, deg, slf):
    c = lax.axis_index("c")
    s = lax.axis_index("s")
    w = s * NC + c
    base = w * EPW
    zeros = jnp.zeros((16,), jnp.float32)
    ones = jnp.ones((16,), jnp.float32)

    def zloop(i, _):
        deg[pl.ds(i * 16, 16)] = zeros
        slf[pl.ds(i * 16, 16)] = zeros
        return ()
    lax.fori_loop(0, NP // 16, zloop, ())

    def chunk(i, _):
        pltpu.sync_copy(src_hbm.at[pl.ds(base + i * CC, CC)], sidx)
        pltpu.sync_copy(dst_hbm.at[pl.ds(base + i * CC, CC)], didx)

        def inner(j, _):
            sv = sidx[pl.ds(j * 16, 16)]
            dv = didx[pl.ds(j * 16, 16)]
            m = sv != dv
            plsc.addupdate_scatter(deg, [dv], ones, mask=m)
            plsc.addupdate_scatter(slf, [dv], ones, mask=jnp.logical_not(m))
            return ()
        lax.fori_loop(0, CC // 16, inner, ())
        return ()
    lax.fori_loop(0, EPW // CC, chunk, ())

    pltpu.sync_copy(deg, out_hbm.at[w, 0])
    pltpu.sync_copy(slf, out_hbm.at[w, 1])


_sc_counts = pl.kernel(
    _counts_body,
    out_type=jax.ShapeDtypeStruct((NW, 2, NP), jnp.float32),
    mesh=_MESH,
    scratch_types=[
        pltpu.VMEM((CC,), jnp.int32),
        pltpu.VMEM((CC,), jnp.int32),
        pltpu.VMEM((NP,), jnp.float32),
        pltpu.VMEM((NP,), jnp.float32),
    ],
    compiler_params=pltpu.CompilerParams(needs_layout_passes=False),
    name="sc_edge_counts",
)


def _agg_body(h_hbm, src_hbm, dst_hbm, out_hbm, sidx, didx, rows0, rows1,
              rows2, acc, isem0, isem1, isem2, gsem0, gsem1, gsem2):
    c = lax.axis_index("c")
    s = lax.axis_index("s")
    w = s * NC + c
    zeros = jnp.zeros((16,), jnp.float32)

    def z1(i, _):
        rows0[i // 8, pl.ds((i % 8) * 16, 16)] = zeros
        return ()
    lax.fori_loop(0, KP * 8, z1, ())

    r0 = s * RPT

    def zc(i, _):
        pltpu.sync_copy(rows0, acc.at[pl.ds(r0 + i * KP, KP)])
        return ()
    lax.fori_loop(0, RPT // KP, zc, ())
    pltpu.sync_copy(rows0.at[pl.ds(0, RPT - KP * (RPT // KP))],
                    acc.at[pl.ds(r0 + KP * (RPT // KP),
                                 RPT - KP * (RPT // KP))])
    plsc.subcore_barrier()

    rows = (rows0, rows1, rows2)
    gsems = (gsem0, gsem1, gsem2)
    isems = (isem0, isem1, isem2)

    def fire_idx(i, slot, sem):
        pltpu.async_copy(src_hbm.at[w, i], sidx.at[slot], sem)
        pltpu.async_copy(dst_hbm.at[w, i], didx.at[slot], sem)

    def wait_idx(slot, sem):
        pltpu.make_async_copy(src_hbm.at[w, 0], sidx.at[slot], sem).wait()
        pltpu.make_async_copy(dst_hbm.at[w, 0], didx.at[slot], sem).wait()

    def fire_gather(slot6, slot3):
        pltpu.async_copy(h_hbm.at[sidx.at[slot6]], rows[slot3], gsems[slot3])

    def wait_gather(slot6, slot3):
        pltpu.make_async_copy(h_hbm.at[sidx.at[slot6]], rows[slot3],
                              gsems[slot3]).wait()

    for i in range(3):
        fire_idx(i, i, isems[i])
    wait_idx(0, isems[0])
    fire_gather(0, 0)
    wait_idx(1, isems[1])
    fire_gather(1, 1)
    fire_idx(3, 3, isems[0])
    fire_idx(4, 4, isems[1])

    def step(ii, _):
        for b in range(GRP):
            i = GRP * ii + b
            b3 = b % 3
            g3 = (b + 2) % 3
            g6 = (b + 2) % GRP
            n6 = (b + 5) % GRP

            @pl.when(i < NCHP)
            def _(i=i, b=b, b3=b3, g3=g3, g6=g6, n6=n6):
                @pl.when(i + 2 < NCHP)
                def _():
                    wait_idx(g6, isems[g3])
                    fire_gather(g6, g3)
                wait_gather(b, b3)
                pltpu.sync_copy(rows[b3], acc.at[didx.at[b]], add=True)

                @pl.when(i + 5 < NCHP)
                def _():
                    fire_idx(i + 5, n6, isems[(b + 5) % 3])
        return ()
    lax.fori_loop(0, (NCHP + GRP - 1) // GRP, step, ())
    plsc.subcore_barrier()

    pltpu.sync_copy(acc.at[pl.ds(r0, RPT)], out_hbm.at[c, pl.ds(r0, RPT)])


_sc_agg = pl.kernel(
    _agg_body,
    out_type=jax.ShapeDtypeStruct((NC, NP, D), jnp.float32),
    mesh=_MESH,
    scratch_types=[
        pltpu.VMEM((GRP, KP), jnp.int32),
        pltpu.VMEM((GRP, KP), jnp.int32),
        pltpu.VMEM((KP, D), jnp.float32),
        pltpu.VMEM((KP, D), jnp.float32),
        pltpu.VMEM((KP, D), jnp.float32),
        pltpu.VMEM_SHARED((NP, D), jnp.float32),
        pltpu.SemaphoreType.DMA,
        pltpu.SemaphoreType.DMA,
        pltpu.SemaphoreType.DMA,
        pltpu.SemaphoreType.DMA,
        pltpu.SemaphoreType.DMA,
        pltpu.SemaphoreType.DMA,
    ],
    name="sc_segment_sum",
)


def _ln_rows(h, s_row, b_row):
    mu = jnp.mean(h, axis=-1, keepdims=True)
    var = jnp.mean((h - mu) ** 2, axis=-1, keepdims=True)
    return (h - mu) / jnp.sqrt(var + EPS) * s_row + b_row


def _in_body(x_ref, w_ref, b_ref, ls_ref, lb_ref, o_ref):
    h = jnp.dot(x_ref[...], w_ref[...], preferred_element_type=jnp.float32)
    h = jnp.maximum(h + b_ref[...], 0.0)
    o_ref[...] = _ln_rows(h, ls_ref[...], lb_ref[...])


def _tc_in(x, W_in, b_in, ln_s, ln_b):
    return pl.pallas_call(
        _in_body,
        grid=(NP // BN,),
        in_specs=[
            pl.BlockSpec((BN, D), lambda i: (i, 0)),
            pl.BlockSpec((D, D), lambda i: (0, 0)),
            pl.BlockSpec((1, D), lambda i: (0, 0)),
            pl.BlockSpec((1, D), lambda i: (0, 0)),
            pl.BlockSpec((1, D), lambda i: (0, 0)),
        ],
        out_specs=pl.BlockSpec((BN, D), lambda i: (i, 0)),
        out_shape=jax.ShapeDtypeStruct((NP, D), jnp.float32),
    )(x, W_in, b_in.reshape(1, D), ln_s.reshape(1, D), ln_b.reshape(1, D))


def _gate_body(h_ref, p0_ref, p1_ref, cnt_ref, w1_ref, w2_ref, bt_ref,
               ls_ref, lb_ref, wo_ref, bo_ref, o_ref, *, fuse_out):
    h = h_ref[...]
    colsum = jnp.sum(cnt_ref[...], axis=0)
    cnt = colsum[0][:, None] + 1.0
    corr = 1.0 - colsum[1][:, None]
    ssum = p0_ref[...] + p1_ref[...] + corr * h
    m = ssum / cnt
    z = (jnp.dot(h, w1_ref[...], preferred_element_type=jnp.float32)
         + jnp.dot(m, w2_ref[...], preferred_element_type=jnp.float32)
         + bt_ref[...])
    tm = jax.nn.sigmoid(z)
    hn = _ln_rows(h * tm + m * (1.0 - tm), ls_ref[...], lb_ref[...])
    if fuse_out:
        o_ref[...] = (jnp.dot(hn, wo_ref[...], preferred_element_type=jnp.float32)
                      + bo_ref[...])
    else:
        o_ref[...] = hn


def _tc_gate(h, p0, p1, cntp, W1, W2, bt, ln_s, ln_b, W_out, b_out, fuse_out):
    return pl.pallas_call(
        functools.partial(_gate_body, fuse_out=fuse_out),
        grid=(NP // BN,),
        in_specs=[
            pl.BlockSpec((BN, D), lambda i: (i, 0)),
            pl.BlockSpec((BN, D), lambda i: (i, 0)),
            pl.BlockSpec((BN, D), lambda i: (i, 0)),
            pl.BlockSpec((NW, 2, BN), lambda i: (0, 0, i)),
            pl.BlockSpec((D, D), lambda i: (0, 0)),
            pl.BlockSpec((D, D), lambda i: (0, 0)),
            pl.BlockSpec((1, D), lambda i: (0, 0)),
            pl.BlockSpec((1, D), lambda i: (0, 0)),
            pl.BlockSpec((1, D), lambda i: (0, 0)),
            pl.BlockSpec((D, D), lambda i: (0, 0)),
            pl.BlockSpec((1, D), lambda i: (0, 0)),
        ],
        out_specs=pl.BlockSpec((BN, D), lambda i: (i, 0)),
        out_shape=jax.ShapeDtypeStruct((NP, D), jnp.float32),
    )(h, p0, p1, cntp, W1, W2, bt.reshape(1, D), ln_s.reshape(1, D),
      ln_b.reshape(1, D), W_out, b_out.reshape(1, D))


def kernel(x, edge_index, W_in, b_in, ln_in_s, ln_in_b, W_tm, b_tm,
           ln1_s, ln1_b, ln2_s, ln2_b, W_out, b_out):
    src = edge_index[0]
    dst = edge_index[1]
    x = jnp.pad(x, ((0, NP - N), (0, 0)))
    padidx = N + (jnp.arange(EPAD - E, dtype=jnp.int32) % (NP - N))
    src_r = jnp.concatenate([src, padidx]).reshape(NW, NCHP, KP)
    dst_r = jnp.concatenate([dst, padidx]).reshape(NW, NCHP, KP)
    rep = D // W_tm.shape[1]
    W_exp = jnp.repeat(W_tm, rep, axis=1)
    W1 = W_exp[:D]
    W2 = W_exp[D:]
    bt = jnp.repeat(b_tm, rep)

    h0 = _tc_in(x, W_in, b_in, ln_in_s, ln_in_b)
    cntp = _sc_counts(src, dst)

    p = _sc_agg(h0, src_r, dst_r)
    h1 = _tc_gate(h0, p[0], p[1], cntp, W1, W2, bt, ln1_s, ln1_b,
                  W_out, b_out, fuse_out=False)
    p2 = _sc_agg(h1, src_r, dst_r)
    out = _tc_gate(h1, p2[0], p2[1], cntp, W1, W2, bt, ln2_s, ln2_b,
                   W_out, b_out, fuse_out=True)
    return out[:N]


# speedup vs baseline: 21.5476x; 1.2046x over previous
import jax
import jax.numpy as jnp
from jax import lax
from jax.experimental import pallas as pl
from jax.experimental.pallas import tpu as pltpu
from jax.experimental.pallas import tpu_sc as plsc

N = 10000
NP = 10240
E = 320000
D = 128
EPS = 1e-5

NC = 2
NS = 16
NW = NC * NS
KP = 128
EPW = 79 * KP
NCH_LAST = (E - 31 * EPW) // KP
GRP = 6
CC = 2048
BN = 1024

_MESH = plsc.VectorSubcoreMesh(
    core_axis_name="c", subcore_axis_name="s", num_cores=NC, num_subcores=NS)


def _counts_body(ei_hbm, out_hbm, idx, deg, slf):
    c = lax.axis_index("c")
    s = lax.axis_index("s")
    w = s * NC + c
    zeros = jnp.zeros((16,), jnp.float32)
    ones = jnp.ones((16,), jnp.float32)

    def zloop(i, _):
        deg[pl.ds(i * 16, 16)] = zeros
        slf[pl.ds(i * 16, 16)] = zeros
        return ()
    lax.fori_loop(0, NP // 16, zloop, ())

    def scan_block(nvec):
        def inner(j, _):
            sv = idx[0, pl.ds(j * 16, 16)]
            dv = idx[1, pl.ds(j * 16, 16)]
            m = sv != dv
            plsc.addupdate_scatter(deg, [dv], ones, mask=m)
            plsc.addupdate_scatter(slf, [dv], ones, mask=jnp.logical_not(m))
            return ()
        lax.fori_loop(0, nvec, inner, ())

    def chunk(i, _):
        pltpu.sync_copy(
            ei_hbm.at[pl.ds(0, 2), pl.ds((i * NW + w) * CC, CC)], idx)
        scan_block(CC // 16)
        return ()
    lax.fori_loop(0, 4, chunk, ())

    @pl.when(w < 28)
    def _():
        pltpu.sync_copy(
            ei_hbm.at[pl.ds(0, 2), pl.ds((4 * NW + w) * CC, CC)], idx)
        scan_block(CC // 16)

    @pl.when(w == 28)
    def _():
        pltpu.sync_copy(
            ei_hbm.at[pl.ds(0, 2), pl.ds((E // CC) * CC, E - (E // CC) * CC)],
            idx.at[pl.ds(0, 2), pl.ds(0, E - (E // CC) * CC)])
        scan_block((E - (E // CC) * CC) // 16)

    pltpu.sync_copy(deg, out_hbm.at[w, 0])
    pltpu.sync_copy(slf, out_hbm.at[w, 1])


_sc_counts = pl.kernel(
    _counts_body,
    out_type=jax.ShapeDtypeStruct((NW, 2, NP), jnp.float32),
    mesh=_MESH,
    scratch_types=[
        pltpu.VMEM((2, CC), jnp.int32),
        pltpu.VMEM((NP,), jnp.float32),
        pltpu.VMEM((NP,), jnp.float32),
    ],
    compiler_params=pltpu.CompilerParams(needs_layout_passes=False),
    name="sc_edge_counts",
)


def _agg_body(h_hbm, ei_hbm, out_hbm, sidx, rows0, rows1, rows2, acc,
              isem0, isem1, isem2, gsem0, gsem1, gsem2, ssem0, ssem1, ssem2):
    c = lax.axis_index("c")
    s = lax.axis_index("s")
    w = s * NC + c
    base = w * EPW
    nch = jnp.where(w == NW - 1, NCH_LAST, EPW // KP)
    zeros = jnp.zeros((16,), jnp.float32)

    r0 = s * 624

    def zero_acc():
        def z1(i, _):
            rows0[i // 8, pl.ds((i % 8) * 16, 16)] = zeros
            return ()
        lax.fori_loop(0, KP * 8, z1, ())

        def zc(i, _):
            pltpu.sync_copy(rows0, acc.at[pl.ds(r0 + i * KP, KP)])
            return ()
        lax.fori_loop(0, 4, zc, ())

        @pl.when(s < NS - 1)
        def _():
            pltpu.sync_copy(rows0.at[pl.ds(0, 624 - 4 * KP)],
                            acc.at[pl.ds(r0 + 4 * KP, 624 - 4 * KP)])

        @pl.when(s == NS - 1)
        def _():
            pltpu.sync_copy(rows0, acc.at[pl.ds(r0 + 4 * KP, KP)])

    rows = (rows0, rows1, rows2)
    gsems = (gsem0, gsem1, gsem2)
    isems = (isem0, isem1, isem2)
    ssems = (ssem0, ssem1, ssem2)

    def fire_idx(i, slot, sem):
        pltpu.async_copy(ei_hbm.at[pl.ds(0, 2), pl.ds(base + i * KP, KP)],
                         sidx.at[slot], sem)

    def wait_idx(slot, sem):
        pltpu.make_async_copy(ei_hbm.at[pl.ds(0, 2), pl.ds(base, KP)],
                              sidx.at[slot], sem).wait()

    def fire_gather(slot6, slot3):
        pltpu.async_copy(h_hbm.at[sidx.at[slot6, 0]], rows[slot3],
                         gsems[slot3])

    def wait_gather(slot6, slot3):
        pltpu.make_async_copy(h_hbm.at[sidx.at[slot6, 0]], rows[slot3],
                              gsems[slot3]).wait()

    def fire_scatter(slot6, slot3):
        pltpu.async_copy(rows[slot3], acc.at[sidx.at[slot6, 1]], ssems[slot3],
                         add=True)

    def wait_scatter(slot3):
        pltpu.make_async_copy(rows[slot3], acc.at[sidx.at[0, 1]],
                              ssems[slot3]).wait()

    for i in range(3):
        fire_idx(i, i, isems[i])
    wait_idx(0, isems[0])
    fire_gather(0, 1)
    wait_idx(1, isems[1])
    fire_gather(1, 2)
    fire_idx(3, 3, isems[0])
    fire_idx(4, 4, isems[1])
    zero_acc()
    plsc.subcore_barrier()

    def step(ii, _):
        for b in range(GRP):
            i = GRP * ii + b
            b3 = (b + 1) % 3
            g3 = b % 3
            gi3 = (b + 2) % 3
            g6 = (b + 2) % GRP
            n6 = (b + 5) % GRP

            @pl.when(i < nch)
            def _(i=i, b=b, b3=b3, g3=g3, gi3=gi3, g6=g6, n6=n6):
                @pl.when(i >= 1)
                def _():
                    wait_scatter(g3)

                @pl.when(i + 2 < nch)
                def _():
                    wait_idx(g6, isems[gi3])
                    fire_gather(g6, g3)
                wait_gather(b, b3)
                fire_scatter(b, b3)

                @pl.when(i + 5 < nch)
                def _():
                    fire_idx(i + 5, n6, isems[(b + 5) % 3])
        return ()
    lax.fori_loop(0, (nch + GRP - 1) // GRP, step, ())
    for k in range(3):
        @pl.when(nch % 3 == k)
        def _(k=k):
            wait_scatter(k)
    plsc.subcore_barrier()

    @pl.when(s < NS - 1)
    def _():
        pltpu.sync_copy(acc.at[pl.ds(r0, 624)],
                        out_hbm.at[pl.ds(c * NP + r0, 624)])

    @pl.when(s == NS - 1)
    def _():
        pltpu.sync_copy(acc.at[pl.ds(r0, 640)],
                        out_hbm.at[pl.ds(c * NP + r0, 640)])


_sc_agg = pl.kernel(
    _agg_body,
    out_type=jax.ShapeDtypeStruct((NC * NP, D), jnp.float32),
    mesh=_MESH,
    scratch_types=[
        pltpu.VMEM((GRP, 2, KP), jnp.int32),
        pltpu.VMEM((KP, D), jnp.float32),
        pltpu.VMEM((KP, D), jnp.float32),
        pltpu.VMEM((KP, D), jnp.float32),
        pltpu.VMEM_SHARED((N, D), jnp.float32),
        pltpu.SemaphoreType.DMA,
        pltpu.SemaphoreType.DMA,
        pltpu.SemaphoreType.DMA,
        pltpu.SemaphoreType.DMA,
        pltpu.SemaphoreType.DMA,
        pltpu.SemaphoreType.DMA,
        pltpu.SemaphoreType.DMA,
        pltpu.SemaphoreType.DMA,
        pltpu.SemaphoreType.DMA,
    ],
    name="sc_segment_sum",
)


def _ln_rows(h, s_row, b_row):
    mu = jnp.mean(h, axis=-1, keepdims=True)
    var = jnp.mean((h - mu) ** 2, axis=-1, keepdims=True)
    return (h - mu) / jnp.sqrt(var + EPS) * s_row + b_row


def _in_body(x_ref, w_ref, b_ref, ls_ref, lb_ref, o_ref):
    h = jnp.dot(x_ref[...], w_ref[...], preferred_element_type=jnp.float32)
    h = jnp.maximum(h + b_ref[...], 0.0)
    o_ref[...] = _ln_rows(h, ls_ref[...], lb_ref[...])


def _tc_in(x, W_in, b_in, ln_s, ln_b):
    return pl.pallas_call(
        _in_body,
        grid=(NP // BN,),
        in_specs=[
            pl.BlockSpec((BN, D), lambda i: (i, 0)),
            pl.BlockSpec((D, D), lambda i: (0, 0)),
            pl.BlockSpec((1, D), lambda i: (0, 0)),
            pl.BlockSpec((1, D), lambda i: (0, 0)),
            pl.BlockSpec((1, D), lambda i: (0, 0)),
        ],
        out_specs=pl.BlockSpec((BN, D), lambda i: (i, 0)),
        out_shape=jax.ShapeDtypeStruct((NP, D), jnp.float32),
    )(x, W_in, b_in.reshape(1, D), ln_s.reshape(1, D), ln_b.reshape(1, D))


def _gate_common(h, p0, p1, cnt, corr, w1, w2, bt, ls, lb):
    ssum = p0 + p1 + corr * h
    m = ssum / cnt
    z = (jnp.dot(h, w1, preferred_element_type=jnp.float32)
         + jnp.dot(m, w2, preferred_element_type=jnp.float32) + bt)
    tm = jax.nn.sigmoid(z)
    return _ln_rows(h * tm + m * (1.0 - tm), ls, lb)


def _gate1_body(h_ref, p0_ref, p1_ref, cntp_ref, w1_ref, w2_ref, bt_ref,
                ls_ref, lb_ref, o_ref, c2_ref):
    colsum = jnp.sum(cntp_ref[...], axis=0)
    cnt_l = colsum[0] + 1.0
    corr_l = 1.0 - colsum[1]
    c2_ref[...] = jnp.stack([cnt_l, corr_l])
    o_ref[...] = _gate_common(h_ref[...], p0_ref[...], p1_ref[...],
                              cnt_l[:, None], corr_l[:, None], w1_ref[...],
                              w2_ref[...], bt_ref[...], ls_ref[...],
                              lb_ref[...])


def _gate2_body(h_ref, p0_ref, p1_ref, c2_ref, w1_ref, w2_ref, bt_ref,
                ls_ref, lb_ref, wo_ref, bo_ref, o_ref):
    hn = _gate_common(h_ref[...], p0_ref[...], p1_ref[...],
                      c2_ref[0][:, None], c2_ref[1][:, None], w1_ref[...],
                      w2_ref[...], bt_ref[...], ls_ref[...], lb_ref[...])
    o_ref[...] = (jnp.dot(hn, wo_ref[...], preferred_element_type=jnp.float32)
                  + bo_ref[...])


def _row_map(i):
    return (i, 0)


def _fix_map(i):
    return (0, 0)


def _tc_gate1(h, p, cntp, W1, W2, bt, ln_s, ln_b):
    return pl.pallas_call(
        _gate1_body,
        grid=(NP // BN,),
        in_specs=[
            pl.BlockSpec((BN, D), _row_map),
            pl.BlockSpec((BN, D), _row_map),
            pl.BlockSpec((BN, D), lambda i: (NP // BN + i, 0)),
            pl.BlockSpec((NW, 2, BN), lambda i: (0, 0, i)),
            pl.BlockSpec((D, D), _fix_map),
            pl.BlockSpec((D, D), _fix_map),
            pl.BlockSpec((1, D), _fix_map),
            pl.BlockSpec((1, D), _fix_map),
            pl.BlockSpec((1, D), _fix_map),
        ],
        out_specs=(pl.BlockSpec((BN, D), _row_map),
                   pl.BlockSpec((2, BN), lambda i: (0, i))),
        out_shape=(jax.ShapeDtypeStruct((NP, D), jnp.float32),
                   jax.ShapeDtypeStruct((2, NP), jnp.float32)),
    )(h, p, p, cntp, W1, W2, bt.reshape(1, D), ln_s.reshape(1, D),
      ln_b.reshape(1, D))


def _tc_gate2(h, p, c2, W1, W2, bt, ln_s, ln_b, W_out, b_out):
    return pl.pallas_call(
        _gate2_body,
        grid=(NP // BN,),
        in_specs=[
            pl.BlockSpec((BN, D), _row_map),
            pl.BlockSpec((BN, D), _row_map),
            pl.BlockSpec((BN, D), lambda i: (NP // BN + i, 0)),
            pl.BlockSpec((2, BN), lambda i: (0, i)),
            pl.BlockSpec((D, D), _fix_map),
            pl.BlockSpec((D, D), _fix_map),
            pl.BlockSpec((1, D), _fix_map),
            pl.BlockSpec((1, D), _fix_map),
            pl.BlockSpec((1, D), _fix_map),
            pl.BlockSpec((D, D), _fix_map),
            pl.BlockSpec((1, D), _fix_map),
        ],
        out_specs=pl.BlockSpec((BN, D), _row_map),
        out_shape=jax.ShapeDtypeStruct((N, D), jnp.float32),
    )(h, p, p, c2, W1, W2, bt.reshape(1, D), ln_s.reshape(1, D),
      ln_b.reshape(1, D), W_out, b_out.reshape(1, D))


def kernel(x, edge_index, W_in, b_in, ln_in_s, ln_in_b, W_tm, b_tm,
           ln1_s, ln1_b, ln2_s, ln2_b, W_out, b_out):
    rep = D // W_tm.shape[1]
    W_exp = jnp.repeat(W_tm, rep, axis=1)
    W1 = W_exp[:D]
    W2 = W_exp[D:]
    bt = jnp.repeat(b_tm, rep)

    h0 = _tc_in(x, W_in, b_in, ln_in_s, ln_in_b)
    cntp = _sc_counts(edge_index)
    h0, cntp = lax.optimization_barrier((h0, cntp))

    p = _sc_agg(h0, edge_index)
    h1, c2 = _tc_gate1(h0, p, cntp, W1, W2, bt, ln1_s, ln1_b)
    p2 = _sc_agg(h1, edge_index)
    return _tc_gate2(h1, p2, c2, W1, W2, bt, ln2_s, ln2_b, W_out, b_out)
